# baseline (speedup 1.0000x reference)
import functools

import jax
import jax.numpy as jnp
from jax import lax
from jax.experimental import pallas as pl
from jax.experimental.pallas import tpu as pltpu
from jax.experimental.pallas import tpu_sc as plsc

_B = 16384
_D = 64
_NC = 2
_NS = 16
_NW = _NC * _NS
_NB = _B // _NW
_C = 128
_NCHUNK = _NB // _C
_L = 16
_MARGIN = 1.0


def _sc_body(idx_hbm, rels_hbm, ent_hbm, rel_hbm, out_hbm,
             hidx, tidx, nhidx, ntidx, ridx,
             h_rows, t_rows, nh_rows, nt_rows, r_rows,
             loss_st, sem):
    wid = lax.axis_index("s") * _NC + lax.axis_index("c")

    def chunk_body(c, lvec):
        base = pl.multiple_of(wid * _NB + c * _C, _C)
        pltpu.sync_copy(idx_hbm.at[pl.ds(0 * _B + base, _C)], hidx)
        pltpu.sync_copy(idx_hbm.at[pl.ds(1 * _B + base, _C)], tidx)
        pltpu.sync_copy(idx_hbm.at[pl.ds(2 * _B + base, _C)], nhidx)
        pltpu.sync_copy(idx_hbm.at[pl.ds(3 * _B + base, _C)], ntidx)
        pltpu.sync_copy(rels_hbm.at[pl.ds(base, _C)], ridx)
        cp1 = pltpu.async_copy(ent_hbm.at[hidx], h_rows, sem)
        cp2 = pltpu.async_copy(ent_hbm.at[tidx], t_rows, sem)
        cp3 = pltpu.async_copy(ent_hbm.at[nhidx], nh_rows, sem)
        cp4 = pltpu.async_copy(ent_hbm.at[ntidx], nt_rows, sem)
        cp5 = pltpu.async_copy(rel_hbm.at[ridx], r_rows, sem)
        cp1.wait(); cp2.wait(); cp3.wait(); cp4.wait(); cp5.wait()

        def g_body(g, lv):
            row = lax.iota(jnp.int32, _L) + g * _L

            def d_body(dd, carry):
                sp, sn = carry
                col = jnp.full((_L,), 0, jnp.int32) + dd
                h = plsc.load_gather(h_rows, [row, col])
                t = plsc.load_gather(t_rows, [row, col])
                r = plsc.load_gather(r_rows, [row, col])
                nh = plsc.load_gather(nh_rows, [row, col])
                nt = plsc.load_gather(nt_rows, [row, col])
                return sp + h * r * t, sn + nh * r * nt

            z = jnp.zeros((_L,), jnp.float32)
            sp, sn = lax.fori_loop(0, _D, d_body, (z, z))
            return lv + jnp.maximum(_MARGIN - sp + sn, 0.0)

        return lax.fori_loop(0, _C // _L, g_body, lvec)

    lvec = lax.fori_loop(0, _NCHUNK, chunk_body, jnp.zeros((_L,), jnp.float32))
    loss_st[...] = lvec
    pltpu.sync_copy(loss_st, out_hbm.at[wid])


@functools.cache
def _make_sc_score():
    return pl.kernel(
        _sc_body,
        out_type=jax.ShapeDtypeStruct((_NW, _L), jnp.float32),
        mesh=plsc.VectorSubcoreMesh(core_axis_name="c", subcore_axis_name="s"),
        compiler_params=pltpu.CompilerParams(
            needs_layout_passes=False, use_tc_tiling_on_sc=False
        ),
        scratch_types=[
            pltpu.VMEM((_C,), jnp.int32),
            pltpu.VMEM((_C,), jnp.int32),
            pltpu.VMEM((_C,), jnp.int32),
            pltpu.VMEM((_C,), jnp.int32),
            pltpu.VMEM((_C,), jnp.int32),
            pltpu.VMEM((_C, _D), jnp.float32),
            pltpu.VMEM((_C, _D), jnp.float32),
            pltpu.VMEM((_C, _D), jnp.float32),
            pltpu.VMEM((_C, _D), jnp.float32),
            pltpu.VMEM((_C, _D), jnp.float32),
            pltpu.VMEM((_L,), jnp.float32),
            pltpu.SemaphoreType.DMA,
        ],
    )


def _reduce_body(x_ref, o_ref):
    o_ref[0, 0] = jnp.sum(x_ref[...]) * (1.0 / _B)


def kernel(pos_pairs, rels, neg_idx, ent_emb, rel_emb):
    idx_flat = jnp.concatenate(
        [pos_pairs[:, 0], pos_pairs[:, 1], neg_idx[0], neg_idx[1]]
    ).astype(jnp.int32)
    partials = _make_sc_score()(idx_flat, rels.astype(jnp.int32), ent_emb, rel_emb)
    loss = pl.pallas_call(
        _reduce_body,
        out_shape=jax.ShapeDtypeStruct((1, 1), jnp.float32),
        out_specs=pl.BlockSpec(memory_space=pltpu.SMEM),
    )(partials)
    return loss[0, 0]


# speedup vs baseline: 1.1013x; 1.1013x over previous
import functools

import jax
import jax.numpy as jnp
from jax import lax
from jax.experimental import pallas as pl
from jax.experimental.pallas import tpu as pltpu
from jax.experimental.pallas import tpu_sc as plsc

_B = 16384
_D = 64
_NC = 2
_NS = 16
_NW = _NC * _NS
_NB = _B // _NW
_C = 64
_NCHUNK = _NB // _C
_L = 16
_MARGIN = 1.0


def _tree_sum(vs):
    while len(vs) > 1:
        vs = [a + b for a, b in zip(vs[::2], vs[1::2])]
    return vs[0]


def _sc_body(pp_hbm, ng_hbm, rels_hbm, ent_hbm, rel_hbm, out_hbm,
             pidx, nhidx, ntidx, ridx,
             pt_rows, nh_rows, nt_rows, r_rows,
             dscratch, loss_st, sem):
    wid = lax.axis_index("s") * _NC + lax.axis_index("c")

    def chunk_body(c, lvec):
        base = pl.multiple_of(wid * _NB + c * _C, _C)
        pltpu.sync_copy(pp_hbm.at[pl.ds(2 * base, 2 * _C)], pidx)
        pltpu.sync_copy(ng_hbm.at[pl.ds(base, _C)], nhidx)
        pltpu.sync_copy(ng_hbm.at[pl.ds(_B + base, _C)], ntidx)
        pltpu.sync_copy(rels_hbm.at[pl.ds(base, _C)], ridx)
        cp1 = pltpu.async_copy(ent_hbm.at[pidx], pt_rows, sem)
        cp2 = pltpu.async_copy(ent_hbm.at[nhidx], nh_rows, sem)
        cp3 = pltpu.async_copy(ent_hbm.at[ntidx], nt_rows, sem)
        cp4 = pltpu.async_copy(rel_hbm.at[ridx], r_rows, sem)
        cp1.wait(); cp2.wait(); cp3.wait(); cp4.wait()

        def e_body(i, carry):
            qs = []
            for k in range(_D // _L):
                sl = pl.ds(_L * k, _L)
                h = pt_rows[2 * i, sl]
                t = pt_rows[2 * i + 1, sl]
                nh = nh_rows[i, sl]
                nt = nt_rows[i, sl]
                r = r_rows[i, sl]
                qs.append((nh * nt - h * t) * r)
            dscratch[pl.ds(i * _L, _L)] = _tree_sum(qs)
            return carry

        lax.fori_loop(0, _C, e_body, 0)

        iota16 = lax.iota(jnp.int32, _L) * _L

        def g_body(g, lv):
            vs = [
                plsc.load_gather(dscratch, [iota16 + (g * (_L * _L) + j)])
                for j in range(_L)
            ]
            return lv + jnp.maximum(_MARGIN + _tree_sum(vs), 0.0)

        return lax.fori_loop(0, _C // _L, g_body, lvec)

    lvec = lax.fori_loop(0, _NCHUNK, chunk_body, jnp.zeros((_L,), jnp.float32))
    loss_st[...] = lvec
    pltpu.sync_copy(loss_st, out_hbm.at[wid])


@functools.cache
def _make_sc_score():
    return pl.kernel(
        _sc_body,
        out_type=jax.ShapeDtypeStruct((_NW, _L), jnp.float32),
        mesh=plsc.VectorSubcoreMesh(core_axis_name="c", subcore_axis_name="s"),
        compiler_params=pltpu.CompilerParams(
            needs_layout_passes=False, use_tc_tiling_on_sc=False
        ),
        scratch_types=[
            pltpu.VMEM((2 * _C,), jnp.int32),
            pltpu.VMEM((_C,), jnp.int32),
            pltpu.VMEM((_C,), jnp.int32),
            pltpu.VMEM((_C,), jnp.int32),
            pltpu.VMEM((2 * _C, _D), jnp.float32),
            pltpu.VMEM((_C, _D), jnp.float32),
            pltpu.VMEM((_C, _D), jnp.float32),
            pltpu.VMEM((_C, _D), jnp.float32),
            pltpu.VMEM((_C * _L,), jnp.float32),
            pltpu.VMEM((_L,), jnp.float32),
            pltpu.SemaphoreType.DMA,
        ],
    )


def _reduce_body(x_ref, o_ref):
    o_ref[0, 0] = jnp.sum(x_ref[...]) * (1.0 / _B)


def kernel(pos_pairs, rels, neg_idx, ent_emb, rel_emb):
    pp = pos_pairs.reshape(-1).astype(jnp.int32)
    ng = neg_idx.reshape(-1).astype(jnp.int32)
    partials = _make_sc_score()(pp, ng, rels.astype(jnp.int32), ent_emb, rel_emb)
    loss = pl.pallas_call(
        _reduce_body,
        out_shape=jax.ShapeDtypeStruct((1, 1), jnp.float32),
        out_specs=pl.BlockSpec(memory_space=pltpu.SMEM),
    )(partials)
    return loss[0, 0]


# speedup vs baseline: 1.7345x; 1.5750x over previous
import functools

import jax
import jax.numpy as jnp
from jax import lax
from jax.experimental import pallas as pl
from jax.experimental.pallas import tpu as pltpu
from jax.experimental.pallas import tpu_sc as plsc

_B = 16384
_D = 64
_NC = 2
_NS = 16
_NW = _NC * _NS
_NB = _B // _NW
_C = 64
_NCHUNK = _NB // _C
_L = 16
_MARGIN = 1.0


def _tree_sum(vs):
    while len(vs) > 1:
        vs = [a + b for a, b in zip(vs[::2], vs[1::2])]
    return vs[0]


def _sc_body(pp_hbm, ng_hbm, rels_hbm, ent_hbm, rel_hbm, out_hbm,
             pp_v, nh_v, nt_v, rl_v,
             hbuf, tbuf, nhbuf, ntbuf, rbuf,
             dscratch, loss_st, sem):
    wid = lax.axis_index("s") * _NC + lax.axis_index("c")

    def chunk_body(c, lvec):
        base = pl.multiple_of(wid * _NB + c * _C, _C)
        pltpu.sync_copy(pp_hbm.at[pl.ds(2 * base, 2 * _C)], pp_v)
        pltpu.sync_copy(ng_hbm.at[pl.ds(base, _C)], nh_v)
        pltpu.sync_copy(ng_hbm.at[pl.ds(_B + base, _C)], nt_v)
        pltpu.sync_copy(rels_hbm.at[pl.ds(base, _C)], rl_v)

        def fire_body(g, carry):
            e0 = g * _L
            pa = pp_v[pl.ds(2 * e0, _L)]
            pb = pp_v[pl.ds(2 * e0 + _L, _L)]
            nhv = nh_v[pl.ds(e0, _L)]
            ntv = nt_v[pl.ds(e0, _L)]
            rlv = rl_v[pl.ds(e0, _L)]
            for m in range(_L):
                i = e0 + m
                src = pa if m < _L // 2 else pb
                eh = src[(2 * m) % _L]
                et = src[(2 * m + 1) % _L]
                pltpu.async_copy(ent_hbm.at[eh], hbuf.at[i], sem)
                pltpu.async_copy(ent_hbm.at[et], tbuf.at[i], sem)
                pltpu.async_copy(ent_hbm.at[nhv[m]], nhbuf.at[i], sem)
                pltpu.async_copy(ent_hbm.at[ntv[m]], ntbuf.at[i], sem)
                pltpu.async_copy(rel_hbm.at[rlv[m]], rbuf.at[i], sem)
            return carry

        lax.fori_loop(0, _C // _L, fire_body, 0)
        for buf in (hbuf, tbuf, nhbuf, ntbuf, rbuf):
            pltpu.make_async_copy(ent_hbm.at[pl.ds(0, _C), :], buf, sem).wait()

        def e_body(i, carry):
            qs = []
            for k in range(_D // _L):
                sl = pl.ds(_L * k, _L)
                h = hbuf[i, sl]
                t = tbuf[i, sl]
                nh = nhbuf[i, sl]
                nt = ntbuf[i, sl]
                r = rbuf[i, sl]
                qs.append((nh * nt - h * t) * r)
            dscratch[pl.ds(i * _L, _L)] = _tree_sum(qs)
            return carry

        lax.fori_loop(0, _C, e_body, 0)

        iota16 = lax.iota(jnp.int32, _L) * _L

        def g_body(g, lv):
            vs = [
                plsc.load_gather(dscratch, [iota16 + (g * (_L * _L) + j)])
                for j in range(_L)
            ]
            return lv + jnp.maximum(_MARGIN + _tree_sum(vs), 0.0)

        return lax.fori_loop(0, _C // _L, g_body, lvec)

    lvec = lax.fori_loop(0, _NCHUNK, chunk_body, jnp.zeros((_L,), jnp.float32))
    loss_st[...] = lvec
    pltpu.sync_copy(loss_st, out_hbm.at[wid])


@functools.cache
def _make_sc_score():
    return pl.kernel(
        _sc_body,
        out_type=jax.ShapeDtypeStruct((_NW, _L), jnp.float32),
        mesh=plsc.VectorSubcoreMesh(core_axis_name="c", subcore_axis_name="s"),
        compiler_params=pltpu.CompilerParams(
            needs_layout_passes=False, use_tc_tiling_on_sc=True
        ),
        scratch_types=[
            pltpu.VMEM((2 * _C,), jnp.int32),
            pltpu.VMEM((_C,), jnp.int32),
            pltpu.VMEM((_C,), jnp.int32),
            pltpu.VMEM((_C,), jnp.int32),
            pltpu.VMEM((_C, _D), jnp.float32),
            pltpu.VMEM((_C, _D), jnp.float32),
            pltpu.VMEM((_C, _D), jnp.float32),
            pltpu.VMEM((_C, _D), jnp.float32),
            pltpu.VMEM((_C, _D), jnp.float32),
            pltpu.VMEM((_C * _L,), jnp.float32),
            pltpu.VMEM((_L,), jnp.float32),
            pltpu.SemaphoreType.DMA,
        ],
    )


def _reduce_body(x_ref, o_ref):
    o_ref[0, 0] = jnp.sum(x_ref[...]) * (1.0 / _B)


def kernel(pos_pairs, rels, neg_idx, ent_emb, rel_emb):
    pp = pos_pairs.reshape(-1).astype(jnp.int32)
    ng = neg_idx.reshape(-1).astype(jnp.int32)
    partials = _make_sc_score()(pp, ng, rels.astype(jnp.int32), ent_emb, rel_emb)
    loss = pl.pallas_call(
        _reduce_body,
        out_shape=jax.ShapeDtypeStruct((1, 1), jnp.float32),
        out_specs=pl.BlockSpec(memory_space=pltpu.SMEM),
    )(partials)
    return loss[0, 0]


# speedup vs baseline: 1.8295x; 1.0548x over previous
import functools

import jax
import jax.numpy as jnp
from jax import lax
from jax.experimental import pallas as pl
from jax.experimental.pallas import tpu as pltpu
from jax.experimental.pallas import tpu_sc as plsc

_B = 16384
_D = 64
_NC = 2
_NS = 16
_NW = _NC * _NS
_NB = _B // _NW
_C = 128
_NCHUNK = _NB // _C
_L = 16
_MARGIN = 1.0


def _tree_sum(vs):
    while len(vs) > 1:
        vs = [a + b for a, b in zip(vs[::2], vs[1::2])]
    return vs[0]


def _sc_body(pp_hbm, ng_hbm, rels_hbm, ent_hbm, rel_hbm, out_hbm,
             pp_v, ng_v, rl_v,
             hbuf, tbuf, nhbuf, ntbuf, rbuf,
             dscratch, loss_st, sem):
    wid = lax.axis_index("s") * _NC + lax.axis_index("c")

    def chunk_body(c, lvec):
        base = pl.multiple_of(wid * _NB + c * _C, _C)
        pltpu.sync_copy(pp_hbm.at[pl.ds(0, 1), pl.ds(base, _C)], pp_v.at[pl.ds(0, 1)])
        pltpu.sync_copy(pp_hbm.at[pl.ds(1, 1), pl.ds(base, _C)], pp_v.at[pl.ds(1, 1)])
        pltpu.sync_copy(ng_hbm.at[pl.ds(0, 1), pl.ds(base, _C)], ng_v.at[pl.ds(0, 1)])
        pltpu.sync_copy(ng_hbm.at[pl.ds(1, 1), pl.ds(base, _C)], ng_v.at[pl.ds(1, 1)])
        pltpu.sync_copy(rels_hbm.at[pl.ds(base, _C)], rl_v)

        def fire_body(g, carry):
            e0 = g * _L
            hvv = pp_v[0, pl.ds(e0, _L)]
            tvv = pp_v[1, pl.ds(e0, _L)]
            nhvv = ng_v[0, pl.ds(e0, _L)]
            ntvv = ng_v[1, pl.ds(e0, _L)]
            rvv = rl_v[pl.ds(e0, _L)]
            for m in range(_L):
                i = e0 + m
                pltpu.async_copy(ent_hbm.at[hvv[m]], hbuf.at[i], sem)
                pltpu.async_copy(ent_hbm.at[tvv[m]], tbuf.at[i], sem)
                pltpu.async_copy(ent_hbm.at[nhvv[m]], nhbuf.at[i], sem)
                pltpu.async_copy(ent_hbm.at[ntvv[m]], ntbuf.at[i], sem)
                pltpu.async_copy(rel_hbm.at[rvv[m]], rbuf.at[i], sem)
            return carry

        lax.fori_loop(0, _C // _L, fire_body, 0)
        for buf in (hbuf, tbuf, nhbuf, ntbuf, rbuf):
            pltpu.make_async_copy(ent_hbm.at[pl.ds(0, _C), :], buf, sem).wait()

        def e_body(i, carry):
            qs = []
            for k in range(_D // _L):
                sl = pl.ds(_L * k, _L)
                h = hbuf[i, sl]
                t = tbuf[i, sl]
                nh = nhbuf[i, sl]
                nt = ntbuf[i, sl]
                r = rbuf[i, sl]
                qs.append((nh * nt - h * t) * r)
            dscratch[pl.ds(i * _L, _L)] = _tree_sum(qs)
            return carry

        lax.fori_loop(0, _C, e_body, 0)

        iota16 = lax.iota(jnp.int32, _L) * _L

        def g_body(g, lv):
            vs = [
                plsc.load_gather(dscratch, [iota16 + (g * (_L * _L) + j)])
                for j in range(_L)
            ]
            return lv + jnp.maximum(_MARGIN + _tree_sum(vs), 0.0)

        return lax.fori_loop(0, _C // _L, g_body, lvec)

    lvec = lax.fori_loop(0, _NCHUNK, chunk_body, jnp.zeros((_L,), jnp.float32))
    loss_st[...] = lvec
    pltpu.sync_copy(loss_st, out_hbm.at[wid])


@functools.cache
def _make_sc_score():
    return pl.kernel(
        _sc_body,
        out_type=jax.ShapeDtypeStruct((_NW, _L), jnp.float32),
        mesh=plsc.VectorSubcoreMesh(core_axis_name="c", subcore_axis_name="s"),
        compiler_params=pltpu.CompilerParams(
            needs_layout_passes=False, use_tc_tiling_on_sc=True
        ),
        scratch_types=[
            pltpu.VMEM((2, _C), jnp.int32),
            pltpu.VMEM((2, _C), jnp.int32),
            pltpu.VMEM((_C,), jnp.int32),
            pltpu.VMEM((_C, _D), jnp.float32),
            pltpu.VMEM((_C, _D), jnp.float32),
            pltpu.VMEM((_C, _D), jnp.float32),
            pltpu.VMEM((_C, _D), jnp.float32),
            pltpu.VMEM((_C, _D), jnp.float32),
            pltpu.VMEM((_C * _L,), jnp.float32),
            pltpu.VMEM((_L,), jnp.float32),
            pltpu.SemaphoreType.DMA,
        ],
    )


def _reduce_body(x_ref, o_ref):
    o_ref[0, 0] = jnp.sum(x_ref[...]) * (1.0 / _B)


def kernel(pos_pairs, rels, neg_idx, ent_emb, rel_emb):
    ppT = pos_pairs.T.astype(jnp.int32)
    ng = neg_idx.astype(jnp.int32)
    partials = _make_sc_score()(ppT, ng, rels.astype(jnp.int32), ent_emb, rel_emb)
    loss = pl.pallas_call(
        _reduce_body,
        out_shape=jax.ShapeDtypeStruct((1, 1), jnp.float32),
        out_specs=pl.BlockSpec(memory_space=pltpu.SMEM),
    )(partials)
    return loss[0, 0]
